# Initial kernel scaffold; baseline (speedup 1.0000x reference)
import jax
import jax.numpy as jnp
from jax.experimental import pallas as pl


def kernel(idx, table):
    raise NotImplementedError("write your pallas kernel here")


# speedup vs baseline: 1.0352x; 1.0352x over previous
import functools

import jax
import jax.numpy as jnp
from jax import lax
from jax.experimental import pallas as pl
from jax.experimental.pallas import tpu as pltpu
from jax.experimental.pallas import tpu_sc as plsc

VOCAB = 1000
B = 1024
T = 50

NC = 2
NS = 16
NW = NC * NS
PER_W = (B * T) // NW
K = 40
NCHUNK = PER_W // K


def _make_gather():
  mesh = plsc.VectorSubcoreMesh(core_axis_name="c", subcore_axis_name="s")

  @functools.partial(
      pl.kernel,
      mesh=mesh,
      compiler_params=pltpu.CompilerParams(use_tc_tiling_on_sc=False),
      out_type=jax.ShapeDtypeStruct((B * T, VOCAB), jnp.float32),
      scratch_types=[
          pltpu.VMEM((NCHUNK, K), jnp.int32),
          pltpu.VMEM((2, K, VOCAB), jnp.float32),
          pltpu.SemaphoreType.DMA,
          pltpu.SemaphoreType.DMA,
      ],
  )
  def gather_kernel(table_hbm, idx_hbm, out_hbm, idx_v, buf, gsem0, gsem1):
    wid = lax.axis_index("s") * NC + lax.axis_index("c")
    base = wid * PER_W
    pltpu.sync_copy(idx_hbm.at[wid], idx_v)
    pltpu.async_copy(table_hbm.at[idx_v.at[0]], buf.at[0], gsem0)
    pltpu.async_copy(table_hbm.at[idx_v.at[1]], buf.at[1], gsem1)

    def body(g, carry):
      j0 = 2 * g
      pltpu.make_async_copy(table_hbm.at[idx_v.at[j0]], buf.at[0], gsem0).wait()
      pltpu.sync_copy(buf.at[0], out_hbm.at[pl.ds(base + j0 * K, K)])

      @pl.when(g < NCHUNK // 2 - 1)
      def _():
        pltpu.async_copy(table_hbm.at[idx_v.at[j0 + 2]], buf.at[0], gsem0)

      pltpu.make_async_copy(
          table_hbm.at[idx_v.at[j0 + 1]], buf.at[1], gsem1).wait()
      pltpu.sync_copy(buf.at[1], out_hbm.at[pl.ds(base + (j0 + 1) * K, K)])

      @pl.when(g < NCHUNK // 2 - 1)
      def _():
        pltpu.async_copy(table_hbm.at[idx_v.at[j0 + 3]], buf.at[1], gsem1)

      return carry

    lax.fori_loop(0, NCHUNK // 2, body, 0)

  return gather_kernel


_gather = jax.jit(_make_gather())


def kernel(idx, table):
  idx_chunks = idx.reshape(NW, NCHUNK, K)
  out = _gather(table, idx_chunks)
  return out.reshape(B, T, VOCAB)
